# baseline (speedup 1.0000x reference)
import functools

import jax
import jax.numpy as jnp
from jax import lax
from jax.experimental import pallas as pl
from jax.experimental.pallas import tpu as pltpu
from jax.experimental.pallas import tpu_sc as plsc

NUM_EMB = 100000
DIM = 128
BATCH = 1024
HIST = 200

TOTAL = BATCH * HIST
CHUNK = 128
NUM_WORKERS = 32
ROWS_PER_W = TOTAL // (NUM_WORKERS * CHUNK)

NBUF = 5
GAHEAD = 3
SLACK = NBUF - GAHEAD
STEADY = ROWS_PER_W - GAHEAD - SLACK


@functools.partial(
    pl.kernel,
    out_type=jax.ShapeDtypeStruct((TOTAL, DIM), jnp.float32),
    mesh=plsc.VectorSubcoreMesh(core_axis_name="c", subcore_axis_name="s"),
    scratch_types=(
        [pltpu.VMEM((ROWS_PER_W, CHUNK), jnp.int32)]
        + [pltpu.VMEM((CHUNK, DIM), jnp.float32) for _ in range(NBUF)]
        + [pltpu.SemaphoreType.DMA for _ in range(2 * NBUF)]
    ),
)
def _gather_kernel(table_hbm, idx_hbm, out_hbm, idx_v, *bufs_and_sems):
    bufs = bufs_and_sems[:NBUF]
    gsems = bufs_and_sems[NBUF:2 * NBUF]
    ssems = bufs_and_sems[2 * NBUF:]

    c = lax.axis_index("c")
    s = lax.axis_index("s")
    wid = s * 2 + c
    row0 = wid * ROWS_PER_W
    pltpu.sync_copy(idx_hbm.at[wid], idx_v)

    def gather_copy(j, b):
        return pltpu.make_async_copy(
            table_hbm.at[idx_v.at[j]], bufs[b], gsems[b])

    def store_copy(j, b):
        return pltpu.make_async_copy(
            bufs[b], out_hbm.at[pl.ds((row0 + j) * CHUNK, CHUNK)], ssems[b])

    for j in range(GAHEAD):
        gather_copy(j, j % NBUF).start()

    for j in range(SLACK):
        gather_copy(j + GAHEAD, (j + GAHEAD) % NBUF).start()
        gather_copy(j, j % NBUF).wait()
        store_copy(j, j % NBUF).start()

    def steady(k, carry):
        j0 = SLACK + k * NBUF
        for b in range(NBUF):
            j = j0 + b
            store_copy(j - SLACK, b).wait()
            gather_copy(j + GAHEAD, b).start()
            bj = (SLACK + b) % NBUF
            gather_copy(j, bj).wait()
            store_copy(j, bj).start()
        return carry

    lax.fori_loop(0, STEADY // NBUF, steady, 0)

    for j in range(ROWS_PER_W - GAHEAD, ROWS_PER_W):
        gather_copy(j, j % NBUF).wait()
        store_copy(j, j % NBUF).start()

    for j in range(ROWS_PER_W - NBUF, ROWS_PER_W):
        store_copy(j, j % NBUF).wait()


def kernel(indices, embedding):
    idx3d = indices.reshape(NUM_WORKERS, ROWS_PER_W, CHUNK).astype(jnp.int32)
    out = _gather_kernel(embedding, idx3d)
    return out.reshape(BATCH, HIST, DIM)
